# initial kernel scaffold (unmeasured)
import jax
import jax.numpy as jnp
from jax import lax
from jax.experimental import pallas as pl
from jax.experimental.pallas import tpu as pltpu

N_DEV = 16
N_TOK = 2048
D_MODEL = 512
D_FF = 1024
N_EXP = 64
EXP_PER_DEV = 4
CHUNK = N_TOK // N_DEV


def kernel(x, router_W, route_idx, expert_W):
    def body(
        x_ref,
        rw_ref,
        idx_ref,
        ew_ref,
        out_ref,
        partial_ref,
        rs_ref,
        ag_ref,
        send_sems,
        recv_sems,
    ):
        my = lax.axis_index("i")
        left = lax.rem(my + N_DEV - 1, N_DEV)
        right = lax.rem(my + 1, N_DEV)

        barrier_sem = pltpu.get_barrier_semaphore()
        for nbr in (left, right):
            pl.semaphore_signal(
                barrier_sem,
                inc=1,
                device_id=(nbr,),
                device_id_type=pl.DeviceIdType.MESH,
            )
        pl.semaphore_wait(barrier_sem, 2)

        xf = x_ref[:, :]
        scores = jnp.dot(xf, rw_ref[:, :], preferred_element_type=jnp.float32)
        m = jnp.max(scores, axis=-1, keepdims=True)
        p = jnp.exp(scores - m)
        probs = p / jnp.sum(p, axis=-1, keepdims=True)

        idx0 = idx_ref[:, 0:1]
        idx1 = idx_ref[:, 1:2]
        ii = lax.broadcasted_iota(jnp.int32, (N_TOK, N_EXP), 1)
        sel = (ii == idx0) | (ii == idx1)
        gated = jnp.where(sel, probs, 0.0)
        gsum = jnp.sum(gated, axis=-1, keepdims=True)

        xb = xf.astype(jnp.bfloat16)
        acc = jnp.zeros((N_TOK, D_FF), jnp.float32)
        for le in range(EXP_PER_DEV):
            e = my * EXP_PER_DEV + le
            w_le = jnp.sum(
                jnp.where(ii == e, gated, 0.0), axis=-1, keepdims=True
            ) / gsum
            wb = ew_ref[le, :, :].astype(jnp.bfloat16)
            y = jnp.dot(xb, wb, preferred_element_type=jnp.float32)
            acc = acc + w_le * y
        partial_ref[:, :] = acc

        own_start = my * CHUNK
        rs_ref[0, :, :] = partial_ref[pl.ds(own_start, CHUNK), :].astype(
            jnp.bfloat16
        )
        for s in range(N_DEV - 1):
            rdma = pltpu.make_async_remote_copy(
                src_ref=rs_ref.at[s],
                dst_ref=rs_ref.at[s + 1],
                send_sem=send_sems.at[s],
                recv_sem=recv_sems.at[s],
                device_id=(right,),
                device_id_type=pl.DeviceIdType.MESH,
            )
            rdma.start()
            rdma.wait()
            rc = lax.rem(my + N_DEV - s - 1, N_DEV)
            pchunk = partial_ref[pl.ds(rc * CHUNK, CHUNK), :]
            rs_ref[s + 1, :, :] = (
                rs_ref[s + 1, :, :].astype(jnp.float32) + pchunk
            ).astype(jnp.bfloat16)


        oc = lax.rem(my + 1, N_DEV)
        ag_ref[0, :, :] = rs_ref[N_DEV - 1, :, :]
        out_ref[pl.ds(oc * CHUNK, CHUNK), :] = rs_ref[N_DEV - 1, :, :].astype(
            jnp.float32
        )
        for t in range(N_DEV - 1):
            rdma = pltpu.make_async_remote_copy(
                src_ref=ag_ref.at[t],
                dst_ref=ag_ref.at[t + 1],
                send_sem=send_sems.at[N_DEV - 1 + t],
                recv_sem=recv_sems.at[N_DEV - 1 + t],
                device_id=(right,),
                device_id_type=pl.DeviceIdType.MESH,
            )
            rdma.start()
            rdma.wait()
            c = lax.rem(my + N_DEV - t, N_DEV)
            out_ref[pl.ds(c * CHUNK, CHUNK), :] = ag_ref[t + 1, :, :].astype(
                jnp.float32
            )

    return pl.pallas_call(
        body,
        out_shape=jax.ShapeDtypeStruct((N_TOK, D_FF), jnp.float32),
        in_specs=[
            pl.BlockSpec(memory_space=pltpu.VMEM),
            pl.BlockSpec(memory_space=pltpu.VMEM),
            pl.BlockSpec(memory_space=pltpu.VMEM),
            pl.BlockSpec(memory_space=pltpu.VMEM),
        ],
        out_specs=pl.BlockSpec(memory_space=pltpu.VMEM),
        scratch_shapes=[
            pltpu.VMEM((N_TOK, D_FF), jnp.float32),
            pltpu.VMEM((N_DEV, CHUNK, D_FF), jnp.bfloat16),
            pltpu.VMEM((N_DEV, CHUNK, D_FF), jnp.bfloat16),
            pltpu.SemaphoreType.DMA((2 * (N_DEV - 1),)),
            pltpu.SemaphoreType.DMA((2 * (N_DEV - 1),)),
        ],
        compiler_params=pltpu.CompilerParams(collective_id=0),
    )(x, router_W, route_idx, expert_W)


# baseline (device time: 165004 ns/iter reference)
import jax
import jax.numpy as jnp
from jax import lax
from jax.experimental import pallas as pl
from jax.experimental.pallas import tpu as pltpu

N_DEV = 16
N_TOK = 2048
D_MODEL = 512
D_FF = 1024
N_EXP = 64
EXP_PER_DEV = 4
CHUNK = N_TOK // N_DEV


def kernel(x, router_W, route_idx, expert_W):
    def body(
        x_ref,
        rw_ref,
        idx_ref,
        ew_ref,
        out_ref,
        partial_ref,
        rs_ref,
        ag_ref,
        send_sems,
        recv_sems,
    ):
        my = lax.axis_index("i")
        left = lax.rem(my + N_DEV - 1, N_DEV)
        right = lax.rem(my + 1, N_DEV)

        barrier_sem = pltpu.get_barrier_semaphore()
        for nbr in (left, right):
            pl.semaphore_signal(
                barrier_sem,
                inc=1,
                device_id=(nbr,),
                device_id_type=pl.DeviceIdType.MESH,
            )
        pl.semaphore_wait(barrier_sem, 2)

        xb = x_ref[:, :].astype(jnp.bfloat16)
        scores = jnp.dot(
            xb,
            rw_ref[:, :].astype(jnp.bfloat16),
            preferred_element_type=jnp.float32,
        )
        m = jnp.max(scores, axis=-1, keepdims=True)
        p = jnp.exp(scores - m)
        probs = p / jnp.sum(p, axis=-1, keepdims=True)

        idx0 = idx_ref[:, 0:1]
        idx1 = idx_ref[:, 1:2]
        ii = lax.broadcasted_iota(jnp.int32, (N_TOK, N_EXP), 1)
        sel = (ii == idx0) | (ii == idx1)
        gated = jnp.where(sel, probs, 0.0)
        gsum = jnp.sum(gated, axis=-1, keepdims=True)

        for le in range(EXP_PER_DEV):
            e = my * EXP_PER_DEV + le
            w_le = jnp.sum(
                jnp.where(ii == e, gated, 0.0), axis=-1, keepdims=True
            ) / gsum
            wb = ew_ref[le, :, :].astype(jnp.bfloat16)
            y = jnp.dot(xb, wb, preferred_element_type=jnp.float32)
            contrib = (w_le * y).astype(jnp.bfloat16)
            if le == 0:
                partial_ref[:, :] = contrib
            else:
                partial_ref[:, :] = partial_ref[:, :] + contrib

        own_start = my * CHUNK
        rs_ref[0, :, :] = partial_ref[pl.ds(own_start, CHUNK), :]
        for s in range(N_DEV - 1):
            rdma = pltpu.make_async_remote_copy(
                src_ref=rs_ref.at[s],
                dst_ref=rs_ref.at[s + 1],
                send_sem=send_sems.at[s],
                recv_sem=recv_sems.at[s],
                device_id=(right,),
                device_id_type=pl.DeviceIdType.MESH,
            )
            rdma.start()
            rdma.wait()
            rc = lax.rem(my + N_DEV - s - 1, N_DEV)
            pchunk = partial_ref[pl.ds(rc * CHUNK, CHUNK), :]
            rs_ref[s + 1, :, :] = rs_ref[s + 1, :, :] + pchunk


        oc = lax.rem(my + 1, N_DEV)
        ag_ref[0, :, :] = rs_ref[N_DEV - 1, :, :]
        out_ref[pl.ds(oc * CHUNK, CHUNK), :] = rs_ref[N_DEV - 1, :, :].astype(
            jnp.float32
        )
        for t in range(N_DEV - 1):
            rdma = pltpu.make_async_remote_copy(
                src_ref=ag_ref.at[t],
                dst_ref=ag_ref.at[t + 1],
                send_sem=send_sems.at[N_DEV - 1 + t],
                recv_sem=recv_sems.at[N_DEV - 1 + t],
                device_id=(right,),
                device_id_type=pl.DeviceIdType.MESH,
            )
            rdma.start()
            rdma.wait()
            c = lax.rem(my + N_DEV - t, N_DEV)
            out_ref[pl.ds(c * CHUNK, CHUNK), :] = ag_ref[t + 1, :, :].astype(
                jnp.float32
            )

    return pl.pallas_call(
        body,
        out_shape=jax.ShapeDtypeStruct((N_TOK, D_FF), jnp.float32),
        in_specs=[
            pl.BlockSpec(memory_space=pltpu.VMEM),
            pl.BlockSpec(memory_space=pltpu.VMEM),
            pl.BlockSpec(memory_space=pltpu.VMEM),
            pl.BlockSpec(memory_space=pltpu.VMEM),
        ],
        out_specs=pl.BlockSpec(memory_space=pltpu.VMEM),
        scratch_shapes=[
            pltpu.VMEM((N_TOK, D_FF), jnp.bfloat16),
            pltpu.VMEM((N_DEV, CHUNK, D_FF), jnp.bfloat16),
            pltpu.VMEM((N_DEV, CHUNK, D_FF), jnp.bfloat16),
            pltpu.SemaphoreType.DMA((2 * (N_DEV - 1),)),
            pltpu.SemaphoreType.DMA((2 * (N_DEV - 1),)),
        ],
        compiler_params=pltpu.CompilerParams(collective_id=0),
    )(x, router_W, route_idx, expert_W)


# device time: 121774 ns/iter; 1.3550x vs baseline; 1.3550x over previous
import jax
import jax.numpy as jnp
from jax import lax
from jax.experimental import pallas as pl
from jax.experimental.pallas import tpu as pltpu

N_DEV = 16
N_TOK = 2048
D_MODEL = 512
D_FF = 1024
N_EXP = 64
EXP_PER_DEV = 4
CHUNK = N_TOK // N_DEV


def kernel(x, router_W, route_idx, expert_W):
    def body(
        x_ref,
        rw_ref,
        idx_ref,
        ew_ref,
        out_ref,
        partial_ref,
        rs_ref,
        ag_ref,
        red_ref,
        rs_send_sems,
        rs_recv_sems,
        ag_send_sems,
        ag_recv_sems,
    ):
        my = lax.axis_index("i")

        barrier_sem = pltpu.get_barrier_semaphore()
        for o in range(1, N_DEV):
            pl.semaphore_signal(
                barrier_sem,
                inc=1,
                device_id=(lax.rem(my + o, N_DEV),),
                device_id_type=pl.DeviceIdType.MESH,
            )
        pl.semaphore_wait(barrier_sem, N_DEV - 1)

        xb = x_ref[:, :].astype(jnp.bfloat16)
        scores = jnp.dot(
            xb,
            rw_ref[:, :].astype(jnp.bfloat16),
            preferred_element_type=jnp.float32,
        )
        m = jnp.max(scores, axis=-1, keepdims=True)
        p = jnp.exp(scores - m)
        probs = p / jnp.sum(p, axis=-1, keepdims=True)

        idx0 = idx_ref[:, 0:1]
        idx1 = idx_ref[:, 1:2]
        ii = lax.broadcasted_iota(jnp.int32, (N_TOK, N_EXP), 1)
        sel = (ii == idx0) | (ii == idx1)
        gated = jnp.where(sel, probs, 0.0)
        gsum = jnp.sum(gated, axis=-1, keepdims=True)

        for le in range(EXP_PER_DEV):
            e = my * EXP_PER_DEV + le
            w_le = jnp.sum(
                jnp.where(ii == e, gated, 0.0), axis=-1, keepdims=True
            ) / gsum
            wb = ew_ref[le, :, :].astype(jnp.bfloat16)
            y = jnp.dot(xb, wb, preferred_element_type=jnp.float32)
            contrib = (w_le * y).astype(jnp.bfloat16)
            if le == 0:
                partial_ref[:, :] = contrib
            else:
                partial_ref[:, :] = partial_ref[:, :] + contrib

        rs_rdmas = []
        for o in range(1, N_DEV):
            t = lax.rem(my + o, N_DEV)
            rdma = pltpu.make_async_remote_copy(
                src_ref=partial_ref.at[pl.ds(t * CHUNK, CHUNK), :],
                dst_ref=rs_ref.at[o - 1],
                send_sem=rs_send_sems.at[o - 1],
                recv_sem=rs_recv_sems.at[o - 1],
                device_id=(t,),
                device_id_type=pl.DeviceIdType.MESH,
            )
            rdma.start()
            rs_rdmas.append(rdma)
        for rdma in rs_rdmas:
            rdma.wait()

        total = partial_ref[pl.ds(my * CHUNK, CHUNK), :].astype(jnp.float32)
        for o in range(1, N_DEV):
            total = total + rs_ref[o - 1, :, :].astype(jnp.float32)
        out_ref[pl.ds(my * CHUNK, CHUNK), :] = total
        red_ref[:, :] = total.astype(jnp.bfloat16)

        ag_rdmas = []
        for o in range(1, N_DEV):
            t = lax.rem(my + o, N_DEV)
            rdma = pltpu.make_async_remote_copy(
                src_ref=red_ref,
                dst_ref=ag_ref.at[o - 1],
                send_sem=ag_send_sems.at[o - 1],
                recv_sem=ag_recv_sems.at[o - 1],
                device_id=(t,),
                device_id_type=pl.DeviceIdType.MESH,
            )
            rdma.start()
            ag_rdmas.append(rdma)
        for o, rdma in enumerate(ag_rdmas, start=1):
            rdma.wait()
            c = lax.rem(my + N_DEV - o, N_DEV)
            out_ref[pl.ds(c * CHUNK, CHUNK), :] = ag_ref[o - 1, :, :].astype(
                jnp.float32
            )

    return pl.pallas_call(
        body,
        out_shape=jax.ShapeDtypeStruct((N_TOK, D_FF), jnp.float32),
        in_specs=[
            pl.BlockSpec(memory_space=pltpu.VMEM),
            pl.BlockSpec(memory_space=pltpu.VMEM),
            pl.BlockSpec(memory_space=pltpu.VMEM),
            pl.BlockSpec(memory_space=pltpu.VMEM),
        ],
        out_specs=pl.BlockSpec(memory_space=pltpu.VMEM),
        scratch_shapes=[
            pltpu.VMEM((N_TOK, D_FF), jnp.bfloat16),
            pltpu.VMEM((N_DEV - 1, CHUNK, D_FF), jnp.bfloat16),
            pltpu.VMEM((N_DEV - 1, CHUNK, D_FF), jnp.bfloat16),
            pltpu.VMEM((CHUNK, D_FF), jnp.bfloat16),
            pltpu.SemaphoreType.DMA((N_DEV - 1,)),
            pltpu.SemaphoreType.DMA((N_DEV - 1,)),
            pltpu.SemaphoreType.DMA((N_DEV - 1,)),
            pltpu.SemaphoreType.DMA((N_DEV - 1,)),
        ],
        compiler_params=pltpu.CompilerParams(collective_id=0),
    )(x, router_W, route_idx, expert_W)


# device time: 87364 ns/iter; 1.8887x vs baseline; 1.3939x over previous
import jax
import jax.numpy as jnp
from jax import lax
from jax.experimental import pallas as pl
from jax.experimental.pallas import tpu as pltpu

N_DEV = 16
N_TOK = 2048
D_MODEL = 512
D_FF = 1024
N_EXP = 64
EXP_PER_DEV = 4
CHUNK = N_TOK // N_DEV
BLOCK = 4 * CHUNK


def kernel(x, router_W, route_idx, expert_W):
    def body(
        x_ref,
        rw_ref,
        idx_ref,
        ew_ref,
        out_ref,
        partial_ref,
        xb_ref,
        ewb_ref,
        wl_ref,
        xcol_ref,
        wcol_ref,
        cps_ref,
        colr_ref,
        prs_ref,
        zrs_ref,
        zag_ref,
        pag_ref,
        send_sems,
        recv_sems,
    ):
        my = lax.axis_index("i")
        p = lax.div(my, 4)
        l = lax.rem(my, 4)

        barrier_sem = pltpu.get_barrier_semaphore()
        lx1 = l + 1 - 2 * lax.rem(l, 2)
        lx3 = 3 - l
        lx2 = 3 - lx1
        plane_peers = [
            4 * p + lx1,
            4 * p + lx3,
            4 * p + lx2,
        ]
        col_peers = [
            4 * lax.rem(p + dz, 4) + l for dz in (1, 2, 3)
        ]
        for nbr in plane_peers + col_peers:
            pl.semaphore_signal(
                barrier_sem,
                inc=1,
                device_id=(nbr,),
                device_id_type=pl.DeviceIdType.MESH,
            )
        pl.semaphore_wait(barrier_sem, 6)

        xb = x_ref[:, :].astype(jnp.bfloat16)
        xb_ref[:, :] = xb
        scores = jnp.dot(
            xb,
            rw_ref[:, :].astype(jnp.bfloat16),
            preferred_element_type=jnp.float32,
        )
        m = jnp.max(scores, axis=-1, keepdims=True)
        pexp = jnp.exp(scores - m)
        probs = pexp / jnp.sum(pexp, axis=-1, keepdims=True)

        idx0 = idx_ref[:, 0:1]
        idx1 = idx_ref[:, 1:2]
        ii = lax.broadcasted_iota(jnp.int32, (N_TOK, N_EXP), 1)
        sel = (ii == idx0) | (ii == idx1)
        gated = jnp.where(sel, probs, 0.0)
        gsum = jnp.sum(gated, axis=-1, keepdims=True)

        for le in range(EXP_PER_DEV):
            e = my * EXP_PER_DEV + le
            wl_ref[:, le : le + 1] = (
                jnp.sum(jnp.where(ii == e, gated, 0.0), axis=-1, keepdims=True)
                / gsum
            )
        for le in range(EXP_PER_DEV):
            ewb_ref[le, :, :] = ew_ref[le, :, :].astype(jnp.bfloat16)

        prs_rdmas = []
        for k, blk in ((2, lx2), (1, lx3), (0, lx1), (None, l)):
            rel = k
            for po in range(4):
                rows = pl.ds((4 * po) * CHUNK + blk * CHUNK, CHUNK)
                xcol_ref[po * CHUNK : (po + 1) * CHUNK, :] = xb_ref[rows, :]
                wcol_ref[po * CHUNK : (po + 1) * CHUNK, :] = wl_ref[rows, :]
            acc = jnp.zeros((BLOCK, D_FF), jnp.float32)
            for le in range(EXP_PER_DEV):
                y = jnp.dot(
                    xcol_ref[:, :],
                    ewb_ref[le, :, :],
                    preferred_element_type=jnp.float32,
                )
                acc = acc + wcol_ref[:, le : le + 1] * y
            partial_ref[pl.ds(blk * BLOCK, BLOCK), :] = acc.astype(jnp.bfloat16)
            if rel is not None:
                rdma = pltpu.make_async_remote_copy(
                    src_ref=partial_ref.at[pl.ds(blk * BLOCK, BLOCK), :],
                    dst_ref=prs_ref.at[k],
                    send_sem=send_sems.at[k],
                    recv_sem=recv_sems.at[k],
                    device_id=(plane_peers[k],),
                    device_id_type=pl.DeviceIdType.MESH,
                )
                rdma.start()
                prs_rdmas.append(rdma)
        for rdma in prs_rdmas:
            rdma.wait()

        cps = (
            partial_ref[pl.ds(l * BLOCK, BLOCK), :].astype(jnp.float32)
            + prs_ref[0, :, :].astype(jnp.float32)
            + prs_ref[1, :, :].astype(jnp.float32)
            + prs_ref[2, :, :].astype(jnp.float32)
        )
        cps_ref[:, :] = cps.astype(jnp.bfloat16)


        zrs_rdmas = []
        for i, dz in enumerate((1, 2, 3)):
            po = lax.rem(p + dz, 4)
            rdma = pltpu.make_async_remote_copy(
                src_ref=cps_ref.at[pl.ds(po * CHUNK, CHUNK), :],
                dst_ref=zrs_ref.at[3 - dz],
                send_sem=send_sems.at[3 + i],
                recv_sem=recv_sems.at[3 + i],
                device_id=(col_peers[i],),
                device_id_type=pl.DeviceIdType.MESH,
            )
            rdma.start()
            zrs_rdmas.append(rdma)
        for rdma in zrs_rdmas:
            rdma.wait()

        red = (
            cps_ref[pl.ds(p * CHUNK, CHUNK), :].astype(jnp.float32)
            + zrs_ref[0, :, :].astype(jnp.float32)
            + zrs_ref[1, :, :].astype(jnp.float32)
            + zrs_ref[2, :, :].astype(jnp.float32)
        )
        out_ref[pl.ds(my * CHUNK, CHUNK), :] = red
        colr_ref[pl.ds(p * CHUNK, CHUNK), :] = red.astype(jnp.bfloat16)

        zag_rdmas = []
        for i, dz in enumerate((1, 2, 3)):
            rdma = pltpu.make_async_remote_copy(
                src_ref=colr_ref.at[pl.ds(p * CHUNK, CHUNK), :],
                dst_ref=zag_ref.at[3 - dz],
                send_sem=send_sems.at[6 + i],
                recv_sem=recv_sems.at[6 + i],
                device_id=(col_peers[i],),
                device_id_type=pl.DeviceIdType.MESH,
            )
            rdma.start()
            zag_rdmas.append(rdma)
        pag_rdmas = []
        for k in range(3):
            rdma = pltpu.make_async_remote_copy(
                src_ref=colr_ref.at[pl.ds(p * CHUNK, CHUNK), :],
                dst_ref=pag_ref.at[k, pl.ds(p * CHUNK, CHUNK), :],
                send_sem=send_sems.at[9 + k],
                recv_sem=recv_sems.at[9 + k],
                device_id=(plane_peers[k],),
                device_id_type=pl.DeviceIdType.MESH,
            )
            rdma.start()
            pag_rdmas.append(rdma)
        for rdma in zag_rdmas:
            rdma.wait()
        for s in range(3):
            po = lax.rem(p + s + 1, 4)
            colr_ref[pl.ds(po * CHUNK, CHUNK), :] = zag_ref[s, :, :]
            out_ref[pl.ds((4 * po) * CHUNK + l * CHUNK, CHUNK), :] = zag_ref[
                s, :, :
            ].astype(jnp.float32)

        for k in range(3):
            for dzi, dz in enumerate((1, 2, 3)):
                po = lax.rem(p + dz, 4)
                rdma = pltpu.make_async_remote_copy(
                    src_ref=colr_ref.at[pl.ds(po * CHUNK, CHUNK), :],
                    dst_ref=pag_ref.at[k, pl.ds(po * CHUNK, CHUNK), :],
                    send_sem=send_sems.at[12 + k * 3 + dzi],
                    recv_sem=recv_sems.at[12 + k * 3 + dzi],
                    device_id=(plane_peers[k],),
                    device_id_type=pl.DeviceIdType.MESH,
                )
                rdma.start()
                pag_rdmas.append(rdma)
        for rdma in pag_rdmas:
            rdma.wait()
        for k in range(3):
            blk = (lx1, lx3, lx2)[k]
            for po in range(4):
                out_ref[pl.ds((4 * po) * CHUNK + blk * CHUNK, CHUNK), :] = (
                    pag_ref[k, pl.ds(po * CHUNK, CHUNK), :].astype(jnp.float32)
                )

    return pl.pallas_call(
        body,
        out_shape=jax.ShapeDtypeStruct((N_TOK, D_FF), jnp.float32),
        in_specs=[
            pl.BlockSpec(memory_space=pltpu.VMEM),
            pl.BlockSpec(memory_space=pltpu.VMEM),
            pl.BlockSpec(memory_space=pltpu.VMEM),
            pl.BlockSpec(memory_space=pltpu.VMEM),
        ],
        out_specs=pl.BlockSpec(memory_space=pltpu.VMEM),
        scratch_shapes=[
            pltpu.VMEM((4 * BLOCK, D_FF), jnp.bfloat16),
            pltpu.VMEM((N_TOK, D_MODEL), jnp.bfloat16),
            pltpu.VMEM((EXP_PER_DEV, D_MODEL, D_FF), jnp.bfloat16),
            pltpu.VMEM((N_TOK, EXP_PER_DEV), jnp.float32),
            pltpu.VMEM((BLOCK, D_MODEL), jnp.bfloat16),
            pltpu.VMEM((BLOCK, EXP_PER_DEV), jnp.float32),
            pltpu.VMEM((BLOCK, D_FF), jnp.bfloat16),
            pltpu.VMEM((BLOCK, D_FF), jnp.bfloat16),
            pltpu.VMEM((3, BLOCK, D_FF), jnp.bfloat16),
            pltpu.VMEM((3, CHUNK, D_FF), jnp.bfloat16),
            pltpu.VMEM((3, CHUNK, D_FF), jnp.bfloat16),
            pltpu.VMEM((3, BLOCK, D_FF), jnp.bfloat16),
            pltpu.SemaphoreType.DMA((21,)),
            pltpu.SemaphoreType.DMA((21,)),
        ],
        compiler_params=pltpu.CompilerParams(collective_id=0),
    )(x, router_W, route_idx, expert_W)
